# baseline (speedup 1.0000x reference)
import jax
import jax.numpy as jnp
from jax.experimental import pallas as pl
from jax.experimental.pallas import tpu as pltpu

NPTS_ = [512, 2048, 8192, 16384, 32768]
BASE_ = [512, 384, 192, 96, 48]
NSEG = 256
HID = 256


def _tc_body(H0, H1, D0, D1, D2, D3, D4, S32, S42, S43, feat0, feat1,
             W0, W1, W2, W3, W4, bb, gg, bebe, out):
    c = jnp.sum(H0[...], axis=2)
    c = jnp.maximum(c, 1.0)

    f0 = feat0[...]
    f1 = feat1[...]
    P0 = jax.lax.dot_general(H0[...][1:].reshape(4 * NSEG, 512), f0,
                             (((1,), (0,)), ((), ())),
                             preferred_element_type=jnp.float32)
    P1 = jax.lax.dot_general(H1[...].reshape(3 * NSEG, 2048), f1,
                             (((1,), (0,)), ((), ())),
                             preferred_element_type=jnp.float32)

    def mm(a, b):
        return jax.lax.dot_general(a, b, (((1,), (0,)), ((), ())),
                                   preferred_element_type=jnp.float32)

    Ws = [W0[...], W1[...], W2[...], W3[...], W4[...]]
    Ds = [D0, D1, D2, D3, D4]

    for i in range(5):
        di = Ds[i][...]
        di = di[0] + di[1]
        w = Ws[i]
        off = BASE_[i]
        pre = mm(di, w[:off])
        if i >= 1:
            if i - 1 == 0:
                src = P0[(i - 1) * NSEG:i * NSEG]
            elif i - 1 == 1:
                src = P1[(i - 2) * NSEG:(i - 1) * NSEG]
            elif i - 1 == 2:
                s = S32 if i == 3 else S42
                src = s[...][0] + s[...][1]
            else:
                src = S43[...][0] + S43[...][1]
            pre = pre + mm(src, w[off:off + BASE_[i - 1]])
            off += BASE_[i - 1]
        if i >= 2:
            if i - 2 == 0:
                src = P0[(i - 1) * NSEG:i * NSEG]
            elif i - 2 == 1:
                src = P1[(i - 2) * NSEG:(i - 1) * NSEG]
            else:
                src = S42[...][0] + S42[...][1]
            pre = pre + mm(src, w[off:off + BASE_[i - 2]])
            off += BASE_[i - 2]
        if i >= 3:
            if i - 3 == 0:
                src = P0[(i - 1) * NSEG:i * NSEG]
            else:
                src = P1[(i - 2) * NSEG:(i - 1) * NSEG]
            pre = pre + mm(src, w[off:off + BASE_[i - 3]])
            off += BASE_[i - 3]
        if i >= 4:
            src = P0[(i - 1) * NSEG:i * NSEG]
            pre = pre + mm(src, w[off:off + BASE_[i - 4]])
            off += BASE_[i - 4]

        pre = pre / c[i][:, None] + bb[...][i][None, :]
        m = jnp.mean(pre, axis=-1, keepdims=True)
        v = jnp.mean((pre - m) * (pre - m), axis=-1, keepdims=True)
        y = (pre - m) * jax.lax.rsqrt(v + 1e-5)
        out[i] = y * gg[...][i][None, :] + bebe[...][i][None, :]


def _tc_call(H0, H1, Ds, S32, S42, S43, feat0, feat1, Ws, bs, gs, bes):
    bb = jnp.stack(bs)
    gg = jnp.stack(gs)
    bebe = jnp.stack(bes)
    out = pl.pallas_call(
        _tc_body,
        out_shape=jax.ShapeDtypeStruct((5, NSEG, HID), jnp.float32),
    )(H0, H1, *Ds, S32, S42, S43, feat0, feat1, *Ws, bb, gg, bebe)
    return out.reshape(5, 1, NSEG, HID)


def kernel(feat0, feat1, feat2, feat3, feat4,
           seg0, seg1, seg2, seg3, seg4,
           inv1, inv2, inv3, inv4,
           W0, W1, W2, W3, W4,
           b0, b1, b2, b3, b4,
           g0, g1, g2, g3, g4,
           be0, be1, be2, be3, be4, max_seg):
    feats = [feat0, feat1, feat2, feat3, feat4]
    segs = [seg0, seg1, seg2, seg3, seg4]
    invs = [None, inv1, inv2, inv3, inv4]

    idx = {}
    for i in range(1, 5):
        idx[(i, i - 1)] = invs[i]
        for j in range(i - 2, -1, -1):
            idx[(i, j)] = invs[j + 1][idx[(i, j + 1)]]

    def hist(seg, ix, n):
        key = seg * n + ix
        return jnp.zeros((NSEG * n,), jnp.float32).at[key].add(1.0).reshape(NSEG, n)

    H0 = jnp.stack([
        hist(seg0, jnp.zeros_like(seg0), 512),
        hist(seg1, idx[(1, 0)], 512),
        hist(seg2, idx[(2, 0)], 512),
        hist(seg3, idx[(3, 0)], 512),
        hist(seg4, idx[(4, 0)], 512),
    ])
    H1 = jnp.stack([
        hist(seg2, idx[(2, 1)], 2048),
        hist(seg3, idx[(3, 1)], 2048),
        hist(seg4, idx[(4, 1)], 2048),
    ])

    def segsum(x, seg):
        s = jax.ops.segment_sum(x, seg, num_segments=NSEG)
        return jnp.stack([s, jnp.zeros_like(s)])

    Ds = [segsum(feats[i], segs[i]) for i in range(5)]
    S32 = segsum(feat2[idx[(3, 2)]], seg3)
    S42 = segsum(feat2[idx[(4, 2)]], seg4)
    S43 = segsum(feat3[idx[(4, 3)]], seg4)

    return _tc_call(H0, H1, Ds, S32, S42, S43, feat0, feat1,
                    [W0, W1, W2, W3, W4],
                    [b0, b1, b2, b3, b4],
                    [g0, g1, g2, g3, g4],
                    [be0, be1, be2, be3, be4])


# speedup vs baseline: 5.4477x; 5.4477x over previous
import functools

import jax
import jax.numpy as jnp
from jax import lax
from jax.experimental import pallas as pl
from jax.experimental.pallas import tpu as pltpu
from jax.experimental.pallas import tpu_sc as plsc

NPTS_ = [512, 2048, 8192, 16384, 32768]
BASE_ = [512, 384, 192, 96, 48]
NSEG = 256
HID = 256

H0SLOT = 128 * 512
H1SLOT = 128 * 2048
H0BASE = 0
H1BASE = 5 * H0SLOT
TRASH = H1BASE + 3 * H1SLOT
HISTSZ = TRASH + 4096


def _sc_body(feat0, feat1, feat2, feat3, feat4,
             seg0, seg1, seg2, seg3, seg4,
             inv1, inv2, inv3, inv4,
             z512, z384, z192, z96, z48, zH,
             d0o, d1o, d2o, d3o, d4o, s32o, s42o, s43o, h0o, h1o,
             A0, A1, A2, A3, A4, AS32, AS42, AS43, HIST,
             rows512, rows384, rows192, rows96, rows48,
             seg8, seg16, seg64, iv64, ix64, seg128,
             ivbuf, ixbuf, flatA, flatB, flat32,
             ones32, ones128, stage, sem):
    c = lax.axis_index("c")
    s = lax.axis_index("s")
    wid = c * 16 + s

    for z, buf, accs in ((z512, rows512, (A0,)), (z384, rows384, (A1,)),
                         (z192, rows192, (A2, AS32, AS42)),
                         (z96, rows96, (A3, AS43)), (z48, rows48, (A4,))):
        stg = buf.at[pl.ds(0, 8)]
        pltpu.sync_copy(z, stg)
        for acc in accs:
            pltpu.sync_copy(stg, acc.at[pl.ds(16 * s, 8)])
            pltpu.sync_copy(stg, acc.at[pl.ds(16 * s + 8, 8)])

    pltpu.sync_copy(zH, stage)

    def zh_body(it, _):
        pltpu.sync_copy(stage, HIST.at[pl.ds(s * 69632 + it * 1024, 1024)])
        return 0
    lax.fori_loop(0, 68, zh_body, 0)
    for k in range(2):
        ones32[pl.ds(16 * k, 16)] = jnp.full((16,), 1.0, jnp.float32)
    for k in range(8):
        ones128[pl.ds(16 * k, 16)] = jnp.full((16,), 1.0, jnp.float32)
    plsc.subcore_barrier()

    def d0_body(it, _):
        cb = 16 * wid + 8 * it
        pltpu.sync_copy(seg0.at[pl.ds(cb, 8)], seg8)
        pltpu.sync_copy(feat0.at[pl.ds(cb, 8)], rows512)
        pltpu.sync_copy(rows512, A0.at[seg8], add=True)
        return 0
    lax.fori_loop(0, 2, d0_body, 0)

    def d1_body(it, _):
        cb = 64 * wid + 8 * it
        pltpu.sync_copy(seg1.at[pl.ds(cb, 8)], seg8)
        pltpu.sync_copy(feat1.at[pl.ds(cb, 8)], rows384)
        pltpu.sync_copy(rows384, A1.at[seg8], add=True)
        return 0
    lax.fori_loop(0, 8, d1_body, 0)

    def diag_chunks(nchunk, seg_h, feat_h, rows, acc, bw):
        sb = seg128 if bw == 128 else seg64

        def body(it, _):
            cb = (wid * nchunk + it) * bw
            pltpu.sync_copy(seg_h.at[pl.ds(cb, bw)], sb)
            pltpu.sync_copy(feat_h.at[pl.ds(cb, bw)], rows)
            pltpu.sync_copy(rows, acc.at[sb], add=True)
            return 0
        lax.fori_loop(0, nchunk, body, 0)

    diag_chunks(4, seg2, feat2, rows192, A2, 64)
    diag_chunks(8, seg3, feat3, rows96, A3, 64)
    diag_chunks(16, seg4, feat4, rows48, A4, 64)

    def s32_body(it, _):
        cb = (wid * 8 + it) * 64
        pltpu.sync_copy(seg3.at[pl.ds(cb, 64)], seg64)
        pltpu.sync_copy(inv3.at[pl.ds(cb, 64)], iv64)
        pltpu.async_copy(feat2.at[iv64], rows192, sem).wait()
        pltpu.sync_copy(rows192, AS32.at[seg64], add=True)
        return 0
    lax.fori_loop(0, 8, s32_body, 0)

    def s43_body(it, _):
        cb = (wid * 16 + it) * 64
        pltpu.sync_copy(seg4.at[pl.ds(cb, 64)], seg64)
        pltpu.sync_copy(inv4.at[pl.ds(cb, 64)], iv64)
        pltpu.async_copy(feat3.at[iv64], rows96, sem).wait()
        pltpu.sync_copy(rows96, AS43.at[seg64], add=True)
        return 0
    lax.fori_loop(0, 16, s43_body, 0)

    def s42_body(it, _):
        cb = (wid * 16 + it) * 64
        pltpu.sync_copy(seg4.at[pl.ds(cb, 64)], seg64)
        pltpu.sync_copy(inv4.at[pl.ds(cb, 64)], iv64)
        pltpu.async_copy(inv3.at[iv64], ix64, sem).wait()
        pltpu.async_copy(feat2.at[ix64], rows192, sem).wait()
        pltpu.sync_copy(rows192, AS42.at[seg64], add=True)
        return 0
    lax.fori_loop(0, 16, s42_body, 0)

    segbase = c * 128

    def histidx(sv, iv, n, slot):
        local = sv - segbase
        inb = (local >= 0) & (local < 128)
        flat = slot + local * n + iv
        return jnp.where(inb, flat, jnp.full((16,), TRASH, jnp.int32))

    pltpu.sync_copy(seg0.at[pl.ds(32 * s, 32)], ivbuf.at[pl.ds(0, 32)])
    for k in range(2):
        sv = ivbuf[pl.ds(16 * k, 16)]
        flat32[pl.ds(16 * k, 16)] = histidx(sv, jnp.zeros((16,), jnp.int32),
                                            512, H0BASE)
    pltpu.sync_copy(ones32, HIST.at[flat32], add=True)

    pltpu.sync_copy(seg1.at[pl.ds(128 * s, 128)], seg128)
    pltpu.sync_copy(inv1.at[pl.ds(128 * s, 128)], ivbuf)
    for k in range(8):
        sv = seg128[pl.ds(16 * k, 16)]
        v10 = ivbuf[pl.ds(16 * k, 16)]
        flatB[pl.ds(16 * k, 16)] = histidx(sv, v10, 512, H0BASE + H0SLOT)
    pltpu.sync_copy(ones128, HIST.at[flatB], add=True)

    def l2_body(it, _):
        cb = s * 512 + it * 128
        pltpu.sync_copy(seg2.at[pl.ds(cb, 128)], seg128)
        pltpu.sync_copy(inv2.at[pl.ds(cb, 128)], ivbuf)
        pltpu.async_copy(inv1.at[ivbuf], ixbuf, sem).wait()
        for k in range(8):
            sv = seg128[pl.ds(16 * k, 16)]
            v21 = ivbuf[pl.ds(16 * k, 16)]
            v20 = ixbuf[pl.ds(16 * k, 16)]
            flatA[pl.ds(16 * k, 16)] = histidx(sv, v21, 2048, H1BASE)
            flatB[pl.ds(16 * k, 16)] = histidx(sv, v20, 512, H0BASE + 2 * H0SLOT)
        pltpu.sync_copy(ones128, HIST.at[flatA], add=True)
        pltpu.sync_copy(ones128, HIST.at[flatB], add=True)
        return 0
    lax.fori_loop(0, 4, l2_body, 0)

    def l3_body(it, _):
        cb = s * 1024 + it * 128
        pltpu.sync_copy(seg3.at[pl.ds(cb, 128)], seg128)
        pltpu.sync_copy(inv3.at[pl.ds(cb, 128)], ivbuf)
        pltpu.async_copy(inv2.at[ivbuf], ixbuf, sem).wait()
        pltpu.async_copy(inv1.at[ixbuf], ivbuf, sem).wait()
        for k in range(8):
            sv = seg128[pl.ds(16 * k, 16)]
            v31 = ixbuf[pl.ds(16 * k, 16)]
            v30 = ivbuf[pl.ds(16 * k, 16)]
            flatA[pl.ds(16 * k, 16)] = histidx(sv, v31, 2048, H1BASE + H1SLOT)
            flatB[pl.ds(16 * k, 16)] = histidx(sv, v30, 512, H0BASE + 3 * H0SLOT)
        pltpu.sync_copy(ones128, HIST.at[flatA], add=True)
        pltpu.sync_copy(ones128, HIST.at[flatB], add=True)
        return 0
    lax.fori_loop(0, 8, l3_body, 0)

    def l4_body(it, _):
        cb = s * 2048 + it * 128
        pltpu.sync_copy(seg4.at[pl.ds(cb, 128)], seg128)
        pltpu.sync_copy(inv4.at[pl.ds(cb, 128)], ivbuf)
        pltpu.async_copy(inv3.at[ivbuf], ixbuf, sem).wait()
        pltpu.async_copy(inv2.at[ixbuf], ivbuf, sem).wait()
        pltpu.async_copy(inv1.at[ivbuf], ixbuf, sem).wait()
        for k in range(8):
            sv = seg128[pl.ds(16 * k, 16)]
            v41 = ivbuf[pl.ds(16 * k, 16)]
            v40 = ixbuf[pl.ds(16 * k, 16)]
            flatA[pl.ds(16 * k, 16)] = histidx(sv, v41, 2048, H1BASE + 2 * H1SLOT)
            flatB[pl.ds(16 * k, 16)] = histidx(sv, v40, 512, H0BASE + 4 * H0SLOT)
        pltpu.sync_copy(ones128, HIST.at[flatA], add=True)
        pltpu.sync_copy(ones128, HIST.at[flatB], add=True)
        return 0
    lax.fori_loop(0, 16, l4_body, 0)

    plsc.subcore_barrier()

    for acc, out, buf in (
            (A0, d0o, rows512),
            (A1, d1o, rows384), (A2, d2o, rows192), (A3, d3o, rows96),
            (A4, d4o, rows48), (AS32, s32o, rows192), (AS42, s42o, rows192),
            (AS43, s43o, rows96)):
        stg = buf.at[pl.ds(0, 8)]
        for h in range(2):
            pltpu.sync_copy(acc.at[pl.ds(16 * s + 8 * h, 8)], stg)
            pltpu.sync_copy(stg, out.at[c, pl.ds(16 * s + 8 * h, 8)])
    for l in range(5):
        src = H0BASE + l * H0SLOT + s * (H0SLOT // 16)
        dst = l * 2 * H0SLOT + c * H0SLOT + s * (H0SLOT // 16)

        def h0_body(it, _, src=src, dst=dst):
            pltpu.sync_copy(HIST.at[pl.ds(src + it * 1024, 1024)], stage)
            pltpu.sync_copy(stage, h0o.at[pl.ds(dst + it * 1024, 1024)])
            return 0
        lax.fori_loop(0, 4, h0_body, 0)
    for l in range(3):
        src = H1BASE + l * H1SLOT + s * (H1SLOT // 16)
        dst = l * 2 * H1SLOT + c * H1SLOT + s * (H1SLOT // 16)

        def h1_body(it, _, src=src, dst=dst):
            pltpu.sync_copy(HIST.at[pl.ds(src + it * 1024, 1024)], stage)
            pltpu.sync_copy(stage, h1o.at[pl.ds(dst + it * 1024, 1024)])
            return 0
        lax.fori_loop(0, 16, h1_body, 0)


def _sc_call(feats, segs, invs):
    f32 = jnp.float32
    i32 = jnp.int32
    mesh = plsc.VectorSubcoreMesh(core_axis_name="c", subcore_axis_name="s")
    out_type = [
        jax.ShapeDtypeStruct((2, NSEG, 512), f32),
        jax.ShapeDtypeStruct((2, NSEG, 384), f32),
        jax.ShapeDtypeStruct((2, NSEG, 192), f32),
        jax.ShapeDtypeStruct((2, NSEG, 96), f32),
        jax.ShapeDtypeStruct((2, NSEG, 48), f32),
        jax.ShapeDtypeStruct((2, NSEG, 192), f32),
        jax.ShapeDtypeStruct((2, NSEG, 192), f32),
        jax.ShapeDtypeStruct((2, NSEG, 96), f32),
        jax.ShapeDtypeStruct((5 * 2 * H0SLOT,), f32),
        jax.ShapeDtypeStruct((3 * 2 * H1SLOT,), f32),
    ]
    scratch = [
        pltpu.VMEM_SHARED((NSEG, 512), f32),
        pltpu.VMEM_SHARED((NSEG, 384), f32),
        pltpu.VMEM_SHARED((NSEG, 192), f32),
        pltpu.VMEM_SHARED((NSEG, 96), f32),
        pltpu.VMEM_SHARED((NSEG, 48), f32),
        pltpu.VMEM_SHARED((NSEG, 192), f32),
        pltpu.VMEM_SHARED((NSEG, 192), f32),
        pltpu.VMEM_SHARED((NSEG, 96), f32),
        pltpu.VMEM_SHARED((HISTSZ,), f32),
        pltpu.VMEM((8, 512), f32),
        pltpu.VMEM((8, 384), f32),
        pltpu.VMEM((64, 192), f32),
        pltpu.VMEM((64, 96), f32),
        pltpu.VMEM((64, 48), f32),
        pltpu.VMEM((8,), i32),
        pltpu.VMEM((16,), i32),
        pltpu.VMEM((64,), i32),
        pltpu.VMEM((64,), i32),
        pltpu.VMEM((64,), i32),
        pltpu.VMEM((128,), i32),
        pltpu.VMEM((128,), i32),
        pltpu.VMEM((128,), i32),
        pltpu.VMEM((128,), i32),
        pltpu.VMEM((128,), i32),
        pltpu.VMEM((32,), i32),
        pltpu.VMEM((32,), f32),
        pltpu.VMEM((128,), f32),
        pltpu.VMEM((1024,), f32),
        pltpu.SemaphoreType.DMA,
    ]
    zs = [jnp.zeros((8, w), f32) for w in (512, 384, 192, 96, 48)]
    zh = jnp.zeros((1024,), f32)
    fn = functools.partial(
        pl.kernel, mesh=mesh, out_type=out_type, scratch_types=scratch,
        compiler_params=pltpu.CompilerParams(needs_layout_passes=False,
                                             use_tc_tiling_on_sc=False),
    )(_sc_body)
    return fn(feats[0], feats[1], feats[2], feats[3], feats[4],
              segs[0], segs[1], segs[2], segs[3], segs[4],
              invs[1], invs[2], invs[3], invs[4], *zs, zh)


def _tc_body(H0, H1, D0, D1, D2, D3, D4, S32, S42, S43, feat0, feat1,
             W0, W1, W2, W3, W4, bb, gg, bebe, out):
    c = jnp.sum(H0[...], axis=2)
    c = jnp.maximum(c, 1.0)

    f0 = feat0[...]
    f1 = feat1[...]
    P0 = jax.lax.dot_general(H0[...][1:].reshape(4 * NSEG, 512), f0,
                             (((1,), (0,)), ((), ())),
                             preferred_element_type=jnp.float32)
    P1 = jax.lax.dot_general(H1[...].reshape(3 * NSEG, 2048), f1,
                             (((1,), (0,)), ((), ())),
                             preferred_element_type=jnp.float32)

    def mm(a, b):
        return jax.lax.dot_general(a, b, (((1,), (0,)), ((), ())),
                                   preferred_element_type=jnp.float32)

    Ws = [W0[...], W1[...], W2[...], W3[...], W4[...]]
    Ds = [D0, D1, D2, D3, D4]

    for i in range(5):
        di = Ds[i][...]
        di = di[0] + di[1]
        w = Ws[i]
        off = BASE_[i]
        pre = mm(di, w[:off])
        if i >= 1:
            if i - 1 == 0:
                src = P0[(i - 1) * NSEG:i * NSEG]
            elif i - 1 == 1:
                src = P1[(i - 2) * NSEG:(i - 1) * NSEG]
            elif i - 1 == 2:
                s = S32 if i == 3 else S42
                src = s[...][0] + s[...][1]
            else:
                src = S43[...][0] + S43[...][1]
            pre = pre + mm(src, w[off:off + BASE_[i - 1]])
            off += BASE_[i - 1]
        if i >= 2:
            if i - 2 == 0:
                src = P0[(i - 1) * NSEG:i * NSEG]
            elif i - 2 == 1:
                src = P1[(i - 2) * NSEG:(i - 1) * NSEG]
            else:
                src = S42[...][0] + S42[...][1]
            pre = pre + mm(src, w[off:off + BASE_[i - 2]])
            off += BASE_[i - 2]
        if i >= 3:
            if i - 3 == 0:
                src = P0[(i - 1) * NSEG:i * NSEG]
            else:
                src = P1[(i - 2) * NSEG:(i - 1) * NSEG]
            pre = pre + mm(src, w[off:off + BASE_[i - 3]])
            off += BASE_[i - 3]
        if i >= 4:
            src = P0[(i - 1) * NSEG:i * NSEG]
            pre = pre + mm(src, w[off:off + BASE_[i - 4]])
            off += BASE_[i - 4]

        pre = pre / c[i][:, None] + bb[...][i][None, :]
        m = jnp.mean(pre, axis=-1, keepdims=True)
        v = jnp.mean((pre - m) * (pre - m), axis=-1, keepdims=True)
        y = (pre - m) * jax.lax.rsqrt(v + 1e-5)
        out[i] = y * gg[...][i][None, :] + bebe[...][i][None, :]


def _tc_call(H0, H1, Ds, S32, S42, S43, feat0, feat1, Ws, bs, gs, bes):
    bb = jnp.stack(bs)
    gg = jnp.stack(gs)
    bebe = jnp.stack(bes)
    out = pl.pallas_call(
        _tc_body,
        out_shape=jax.ShapeDtypeStruct((5, NSEG, HID), jnp.float32),
    )(H0, H1, *Ds, S32, S42, S43, feat0, feat1, *Ws, bb, gg, bebe)
    return out.reshape(5, 1, NSEG, HID)


def kernel(feat0, feat1, feat2, feat3, feat4,
           seg0, seg1, seg2, seg3, seg4,
           inv1, inv2, inv3, inv4,
           W0, W1, W2, W3, W4,
           b0, b1, b2, b3, b4,
           g0, g1, g2, g3, g4,
           be0, be1, be2, be3, be4, max_seg):
    feats = [feat0, feat1, feat2, feat3, feat4]
    segs = [seg0, seg1, seg2, seg3, seg4]
    invs = [None, inv1, inv2, inv3, inv4]

    (D0, D1, D2, D3, D4, S32, S42, S43, H0f, H1f) = _sc_call(feats, segs, invs)
    Ds = [D0, D1, D2, D3, D4]
    H0 = H0f.reshape(5, NSEG, 512)
    H1 = H1f.reshape(3, NSEG, 2048)

    return _tc_call(H0, H1, Ds, S32, S42, S43, feat0, feat1,
                    [W0, W1, W2, W3, W4],
                    [b0, b1, b2, b3, b4],
                    [g0, g1, g2, g3, g4],
                    [be0, be1, be2, be3, be4])


# speedup vs baseline: 7.7300x; 1.4189x over previous
import functools

import jax
import jax.numpy as jnp
from jax import lax
from jax.experimental import pallas as pl
from jax.experimental.pallas import tpu as pltpu
from jax.experimental.pallas import tpu_sc as plsc

NPTS_ = [512, 2048, 8192, 16384, 32768]
BASE_ = [512, 384, 192, 96, 48]
NSEG = 256
HID = 256

H0SLOT = 128 * 512
H1SLOT = 128 * 2048
H0BASE = 0
H1BASE = 5 * H0SLOT
TRASH = H1BASE + 3 * H1SLOT
HISTSZ = TRASH + 4096


def _sc_body(feat2, feat3,
             seg0, seg1, seg2, seg3, seg4,
             inv1, inv2, inv3, inv4,
             z192, z96, zh,
             s32o, s42o, s43o, h0o, h1o,
             AS32, AS42, AS43, HIST,
             rows192, rows96,
             sega, segb, iva, ivb, ixa,
             seg2k, iv2k, ix2k, flat2d,
             ones128, stage, semg, semh, sems):
    c = lax.axis_index("c")
    s = lax.axis_index("s")
    wid = c * 16 + s

    for z, buf, accs in ((z192, rows192, (AS32, AS42)),
                         (z96, rows96, (AS43,))):
        stg = buf.at[pl.ds(0, 8)]
        pltpu.sync_copy(z, stg)
        for acc in accs:
            pltpu.sync_copy(stg, acc.at[pl.ds(16 * s, 8)])
            pltpu.sync_copy(stg, acc.at[pl.ds(16 * s + 8, 8)])

    pltpu.sync_copy(zh, stage)

    def zh_body(it, _):
        pltpu.sync_copy(stage, HIST.at[pl.ds(s * 69632 + it * 2048, 2048)])
        return 0
    lax.fori_loop(0, 34, zh_body, 0)
    for k in range(8):
        ones128[pl.ds(16 * k, 16)] = jnp.full((16,), 1.0, jnp.float32)
    plsc.subcore_barrier()

    def s_pair(nch, seg_h, iv_h, hop_h, table, rows, acc):
        scat = [None, None]
        for i in range(nch):
            h = i % 2
            if scat[h] is not None:
                scat[h].wait()
            cb = (wid * nch + i) * 64
            segr = sega if h == 0 else segb
            ivr = iva if h == 0 else ivb
            half = rows.at[pl.ds(h * 64, 64)]
            cp1 = pltpu.async_copy(seg_h.at[pl.ds(cb, 64)], segr, semg)
            cp2 = pltpu.async_copy(iv_h.at[pl.ds(cb, 64)], ivr, semg)
            cp1.wait()
            cp2.wait()
            if hop_h is not None:
                pltpu.async_copy(hop_h.at[ivr], ixa, semg).wait()
                ivr = ixa
            pltpu.async_copy(table.at[ivr], half, semg).wait()
            scat[h] = pltpu.async_copy(half, acc.at[segr], sems, add=True)
        for cp in scat:
            if cp is not None:
                cp.wait()

    s_pair(8, seg3, inv3, None, feat2, rows192, AS32)
    s_pair(16, seg4, inv4, None, feat3, rows96, AS43)
    s_pair(16, seg4, inv4, inv3, feat2, rows192, AS42)

    segbase = c * 128

    def histidx(sv, iv, n, slot):
        local = sv - segbase
        inb = (local >= 0) & (local < 128)
        flat = slot + local * n + iv
        return jnp.where(inb, flat, jnp.full((16,), TRASH, jnp.int32))

    hist_cps = []
    frow = [0]

    def scatter_chunks(npts, specs):
        for j in range(npts // 128):
            for ivref, n, slot in specs:
                r = frow[0]
                frow[0] += 1
                for k in range(8):
                    sv = seg2k[pl.ds(j * 128 + 16 * k, 16)]
                    va = ivref[pl.ds(j * 128 + 16 * k, 16)]
                    flat2d[r, pl.ds(16 * k, 16)] = histidx(sv, va, n, slot)
                hist_cps.append(pltpu.async_copy(
                    ones128, HIST.at[flat2d.at[r]], semh, add=True))

    pltpu.sync_copy(seg0.at[pl.ds(32 * s, 32)], seg2k.at[pl.ds(0, 32)])
    r0 = frow[0]
    zid = jnp.zeros((16,), jnp.int32)
    for k in range(8):
        if k < 2:
            sv = seg2k[pl.ds(16 * k, 16)]
            flat2d[r0, pl.ds(16 * k, 16)] = histidx(sv, zid, 512, H0BASE)
        else:
            flat2d[r0, pl.ds(16 * k, 16)] = jnp.full((16,), TRASH, jnp.int32)
    hist_cps.append(pltpu.async_copy(
        ones128, HIST.at[flat2d.at[r0]], semh, add=True))
    frow[0] += 1

    pltpu.sync_copy(seg1.at[pl.ds(128 * s, 128)], seg2k.at[pl.ds(0, 128)])
    pltpu.sync_copy(inv1.at[pl.ds(128 * s, 128)], iv2k.at[pl.ds(0, 128)])
    scatter_chunks(128, [(iv2k, 512, H0BASE + H0SLOT)])

    pltpu.sync_copy(seg2.at[pl.ds(512 * s, 512)], seg2k.at[pl.ds(0, 512)])
    pltpu.sync_copy(inv2.at[pl.ds(512 * s, 512)], iv2k.at[pl.ds(0, 512)])
    pltpu.async_copy(inv1.at[iv2k.at[pl.ds(0, 512)]],
                     ix2k.at[pl.ds(0, 512)], semg).wait()
    scatter_chunks(512, [(iv2k, 2048, H1BASE),
                         (ix2k, 512, H0BASE + 2 * H0SLOT)])

    pltpu.sync_copy(seg3.at[pl.ds(1024 * s, 1024)], seg2k.at[pl.ds(0, 1024)])
    pltpu.sync_copy(inv3.at[pl.ds(1024 * s, 1024)], iv2k.at[pl.ds(0, 1024)])
    pltpu.async_copy(inv2.at[iv2k.at[pl.ds(0, 1024)]],
                     ix2k.at[pl.ds(0, 1024)], semg).wait()
    pltpu.async_copy(inv1.at[ix2k.at[pl.ds(0, 1024)]],
                     iv2k.at[pl.ds(0, 1024)], semg).wait()
    scatter_chunks(1024, [(ix2k, 2048, H1BASE + H1SLOT),
                          (iv2k, 512, H0BASE + 3 * H0SLOT)])

    pltpu.sync_copy(seg4.at[pl.ds(2048 * s, 2048)], seg2k)
    pltpu.sync_copy(inv4.at[pl.ds(2048 * s, 2048)], iv2k)
    pltpu.async_copy(inv3.at[iv2k], ix2k, semg).wait()
    pltpu.async_copy(inv2.at[ix2k], iv2k, semg).wait()
    scatter_chunks(2048, [(iv2k, 2048, H1BASE + 2 * H1SLOT)])
    pltpu.async_copy(inv1.at[iv2k], ix2k, semg).wait()
    scatter_chunks(2048, [(ix2k, 512, H0BASE + 4 * H0SLOT)])

    for cp in hist_cps:
        cp.wait()
    plsc.subcore_barrier()

    for acc, out, buf in ((AS32, s32o, rows192), (AS42, s42o, rows192),
                          (AS43, s43o, rows96)):
        stg = buf.at[pl.ds(0, 8)]
        for h in range(2):
            pltpu.sync_copy(acc.at[pl.ds(16 * s + 8 * h, 8)], stg)
            pltpu.sync_copy(stg, out.at[c, pl.ds(16 * s + 8 * h, 8)])
    for l in range(5):
        src = H0BASE + l * H0SLOT + s * (H0SLOT // 16)
        dst = l * 2 * H0SLOT + c * H0SLOT + s * (H0SLOT // 16)
        def h0_body(it, _, src=src, dst=dst):
            pltpu.sync_copy(HIST.at[pl.ds(src + it * 2048, 2048)], stage)
            pltpu.sync_copy(stage, h0o.at[pl.ds(dst + it * 2048, 2048)])
            return 0
        lax.fori_loop(0, 2, h0_body, 0)
    for l in range(3):
        src = H1BASE + l * H1SLOT + s * (H1SLOT // 16)
        dst = l * 2 * H1SLOT + c * H1SLOT + s * (H1SLOT // 16)

        def h1_body(it, _, src=src, dst=dst):
            pltpu.sync_copy(HIST.at[pl.ds(src + it * 2048, 2048)], stage)
            pltpu.sync_copy(stage, h1o.at[pl.ds(dst + it * 2048, 2048)])
            return 0
        lax.fori_loop(0, 8, h1_body, 0)


def _sc_call(feats, segs, invs):
    f32 = jnp.float32
    i32 = jnp.int32
    mesh = plsc.VectorSubcoreMesh(core_axis_name="c", subcore_axis_name="s")
    out_type = [
        jax.ShapeDtypeStruct((2, NSEG, 192), f32),
        jax.ShapeDtypeStruct((2, NSEG, 192), f32),
        jax.ShapeDtypeStruct((2, NSEG, 96), f32),
        jax.ShapeDtypeStruct((5 * 2 * H0SLOT,), f32),
        jax.ShapeDtypeStruct((3 * 2 * H1SLOT,), f32),
    ]
    scratch = [
        pltpu.VMEM_SHARED((NSEG, 192), f32),
        pltpu.VMEM_SHARED((NSEG, 192), f32),
        pltpu.VMEM_SHARED((NSEG, 96), f32),
        pltpu.VMEM_SHARED((HISTSZ,), f32),
        pltpu.VMEM((128, 192), f32),
        pltpu.VMEM((128, 96), f32),
        pltpu.VMEM((64,), i32),
        pltpu.VMEM((64,), i32),
        pltpu.VMEM((64,), i32),
        pltpu.VMEM((64,), i32),
        pltpu.VMEM((64,), i32),
        pltpu.VMEM((2048,), i32),
        pltpu.VMEM((2048,), i32),
        pltpu.VMEM((2048,), i32),
        pltpu.VMEM((58, 128), i32),
        pltpu.VMEM((128,), f32),
        pltpu.VMEM((2048,), f32),
        pltpu.SemaphoreType.DMA,
        pltpu.SemaphoreType.DMA,
        pltpu.SemaphoreType.DMA,
    ]
    zs = [jnp.zeros((8, w), f32) for w in (192, 96)]
    zh = jnp.zeros((2048,), f32)
    fn = functools.partial(
        pl.kernel, mesh=mesh, out_type=out_type, scratch_types=scratch,
        compiler_params=pltpu.CompilerParams(needs_layout_passes=False,
                                             use_tc_tiling_on_sc=False),
    )(_sc_body)
    return fn(feats[2], feats[3],
              segs[0], segs[1], segs[2], segs[3], segs[4],
              invs[1], invs[2], invs[3], invs[4], *zs, zh)


def _tc_body(H0, H1, S32, S42, S43,
             feat0, feat1, feat2, feat3, feat4,
             sg0, sg1, sg2, sg3, sg4,
             W0, W1, W2, W3, W4, bb, gg, bebe, out):
    def mm(a, b):
        return jax.lax.dot_general(a, b, (((1,), (0,)), ((), ())),
                                   preferred_element_type=jnp.float32)

    def mmT(a, b):
        return jax.lax.dot_general(a, b, (((0,), (0,)), ((), ())),
                                   preferred_element_type=jnp.float32)

    def diag(seg_ref, feat_ref, npts, width, grp):
        nrow = npts // 128

        def body(g, acc):
            ohs = []
            for u in range(grp):
                sv = seg_ref[pl.ds(g * grp + u, 1), :]
                ohs.append((lax.broadcasted_iota(jnp.int32, (NSEG, 128), 0)
                            == sv).astype(jnp.float32))
            oh = jnp.concatenate(ohs, axis=1)
            f = feat_ref[pl.ds(g * grp * 128, grp * 128), :]
            return acc + mm(oh, f)
        return lax.fori_loop(0, nrow // grp, body,
                             jnp.zeros((NSEG, width), jnp.float32))

    Ds = [diag(sg0, feat0, 512, 512, 4),
          diag(sg1, feat1, 2048, 384, 16),
          diag(sg2, feat2, 8192, 192, 16),
          diag(sg3, feat3, 16384, 96, 16),
          diag(sg4, feat4, 32768, 48, 16)]

    c = jnp.sum(H0[...], axis=2)
    c = jnp.maximum(c, 1.0)

    P0 = mm(H0[...][1:].reshape(4 * NSEG, 512), feat0[...])
    P1 = mm(H1[...].reshape(3 * NSEG, 2048), feat1[...])

    Ws = [W0[...], W1[...], W2[...], W3[...], W4[...]]
    Ss = {(3, 2): S32, (4, 2): S42, (4, 3): S43}

    for i in range(5):
        w = Ws[i]
        off = BASE_[i]
        pre = mm(Ds[i], w[:off])
        for j in range(i - 1, -1, -1):
            if j == 0:
                src = P0[(i - 1) * NSEG:i * NSEG]
            elif j == 1:
                src = P1[(i - 2) * NSEG:(i - 1) * NSEG]
            else:
                sp = Ss[(i, j)][...]
                src = sp[0] + sp[1]
            pre = pre + mm(src, w[off:off + BASE_[j]])
            off += BASE_[j]

        pre = pre / c[i][:, None] + bb[...][i][None, :]
        m = jnp.mean(pre, axis=-1, keepdims=True)
        v = jnp.mean((pre - m) * (pre - m), axis=-1, keepdims=True)
        y = (pre - m) * jax.lax.rsqrt(v + 1e-5)
        out[i] = y * gg[...][i][None, :] + bebe[...][i][None, :]


def _tc_call(H0, H1, S32, S42, S43, feats, segs, Ws, bs, gs, bes):
    bb = jnp.stack(bs)
    gg = jnp.stack(gs)
    bebe = jnp.stack(bes)
    sgs = [s.reshape(-1, 128) for s in segs]
    out = pl.pallas_call(
        _tc_body,
        out_shape=jax.ShapeDtypeStruct((5, NSEG, HID), jnp.float32),
    )(H0, H1, S32, S42, S43, *feats, *sgs, *Ws, bb, gg, bebe)
    return out.reshape(5, 1, NSEG, HID)


def kernel(feat0, feat1, feat2, feat3, feat4,
           seg0, seg1, seg2, seg3, seg4,
           inv1, inv2, inv3, inv4,
           W0, W1, W2, W3, W4,
           b0, b1, b2, b3, b4,
           g0, g1, g2, g3, g4,
           be0, be1, be2, be3, be4, max_seg):
    feats = [feat0, feat1, feat2, feat3, feat4]
    segs = [seg0, seg1, seg2, seg3, seg4]
    invs = [None, inv1, inv2, inv3, inv4]

    (S32, S42, S43, H0f, H1f) = _sc_call(feats, segs, invs)
    H0 = H0f.reshape(5, NSEG, 512)
    H1 = H1f.reshape(3, NSEG, 2048)

    return _tc_call(H0, H1, S32, S42, S43, feats, segs,
                    [W0, W1, W2, W3, W4],
                    [b0, b1, b2, b3, b4],
                    [g0, g1, g2, g3, g4],
                    [be0, be1, be2, be3, be4])
